# Initial kernel scaffold; baseline (speedup 1.0000x reference)
import functools

import jax
import jax.numpy as jnp
from jax import lax
from jax.experimental import pallas as pl
from jax.experimental.pallas import tpu as pltpu
from jax.experimental.pallas import tpu_sc as plsc

D = 128
L = 16
NC = 2
NS = 16
NW = NC * NS
C = 80


@functools.partial(jax.jit, static_argnums=(3,))
def _run(src_idx, dst_idx, z, E):
    e_per_w = E // NW
    n_chunks = e_per_w // C
    mesh = plsc.VectorSubcoreMesh(core_axis_name="c", subcore_axis_name="s")

    @functools.partial(
        pl.kernel,
        mesh=mesh,
        out_type=jax.ShapeDtypeStruct((E,), jnp.float32),
        scratch_types=[
            pltpu.VMEM((e_per_w,), jnp.int32),
            pltpu.VMEM((e_per_w,), jnp.int32),
            pltpu.VMEM((C, D), jnp.float32),
            pltpu.VMEM((C, D), jnp.float32),
            pltpu.VMEM((e_per_w,), jnp.float32),
            pltpu.SemaphoreType.DMA,
        ],
    )
    def k(src_hbm, dst_hbm, z_hbm, out_hbm, idx_s, idx_d, rows_s, rows_d,
          out_v, sem):
        cid = lax.axis_index("c")
        sid = lax.axis_index("s")
        wid = sid * NC + cid
        base = wid * e_per_w

        pltpu.sync_copy(src_hbm.at[pl.ds(base, e_per_w)], idx_s)
        pltpu.sync_copy(dst_hbm.at[pl.ds(base, e_per_w)], idx_d)

        def chunk_body(i, carry):
            off = i * C
            cp_s = pltpu.async_copy(z_hbm.at[idx_s.at[pl.ds(off, C)]],
                                    rows_s, sem)
            cp_d = pltpu.async_copy(z_hbm.at[idx_d.at[pl.ds(off, C)]],
                                    rows_d, sem)
            cp_s.wait()
            cp_d.wait()

            def group_body(g, carry2):
                rid = lax.iota(jnp.int32, L) + g * L
                acc = jnp.zeros((L,), jnp.float32)
                for f in range(D):
                    col = jnp.full((L,), f, jnp.int32)
                    sv = plsc.load_gather(rows_s, [rid, col])
                    dv = plsc.load_gather(rows_d, [rid, col])
                    acc = acc + sv * dv
                sig = 1.0 / (1.0 + jnp.exp(-acc))
                out_v[pl.ds(off + g * L, L)] = sig
                return carry2

            lax.fori_loop(0, C // L, group_body, 0)
            return carry

        lax.fori_loop(0, n_chunks, chunk_body, 0)
        pltpu.sync_copy(out_v, out_hbm.at[pl.ds(base, e_per_w)])

    return k(src_idx, dst_idx, z)


def kernel(z, edge_index):
    idx = edge_index.astype(jnp.int32)
    return _run(idx[0], idx[1], z, idx.shape[1])


# speedup vs baseline: 3.4376x; 3.4376x over previous
import functools

import jax
import jax.numpy as jnp
from jax import lax
from jax.experimental import pallas as pl
from jax.experimental.pallas import tpu as pltpu
from jax.experimental.pallas import tpu_sc as plsc

D = 128
L = 16
NC = 2
NS = 16
NW = NC * NS
C = 80


@functools.partial(jax.jit, static_argnums=(3,))
def _run(src_idx, dst_idx, z, E):
    e_per_w = E // NW
    n_chunks = e_per_w // C
    mesh = plsc.VectorSubcoreMesh(core_axis_name="c", subcore_axis_name="s")

    @functools.partial(
        pl.kernel,
        mesh=mesh,
        compiler_params=pltpu.CompilerParams(needs_layout_passes=False),
        out_type=jax.ShapeDtypeStruct((E,), jnp.float32),
        scratch_types=[
            pltpu.VMEM((e_per_w,), jnp.int32),
            pltpu.VMEM((e_per_w,), jnp.int32),
            pltpu.VMEM((C, D), jnp.float32),
            pltpu.VMEM((C, D), jnp.float32),
            pltpu.VMEM((e_per_w,), jnp.float32),
            pltpu.SemaphoreType.DMA,
        ],
    )
    def k(src_hbm, dst_hbm, z_hbm, out_hbm, idx_s, idx_d, rows_s, rows_d,
          out_v, sem):
        cid = lax.axis_index("c")
        sid = lax.axis_index("s")
        wid = sid * NC + cid
        base = wid * e_per_w

        pltpu.sync_copy(src_hbm.at[pl.ds(base, e_per_w)], idx_s)
        pltpu.sync_copy(dst_hbm.at[pl.ds(base, e_per_w)], idx_d)

        def chunk_body(i, carry):
            off = i * C
            cp_s = pltpu.async_copy(z_hbm.at[idx_s.at[pl.ds(off, C)]],
                                    rows_s, sem)
            cp_d = pltpu.async_copy(z_hbm.at[idx_d.at[pl.ds(off, C)]],
                                    rows_d, sem)
            cp_s.wait()
            cp_d.wait()

            lane = lax.iota(jnp.int32, L)

            def group_body(g, carry2):
                e0 = g * L
                res = jnp.zeros((L,), jnp.float32)
                for j in range(L):
                    e = e0 + j
                    acc = jnp.zeros((L,), jnp.float32)
                    for kk in range(D // L):
                        sv = rows_s[e, pl.ds(kk * L, L)]
                        dv = rows_d[e, pl.ds(kk * L, L)]
                        acc = acc + sv * dv
                    dot = jnp.sum(acc)
                    res = jnp.where(lane == j, dot, res)
                sig = 1.0 / (1.0 + jnp.exp(-res))
                out_v[pl.ds(off + g * L, L)] = sig
                return carry2

            lax.fori_loop(0, C // L, group_body, 0)
            return carry

        lax.fori_loop(0, n_chunks, chunk_body, 0)
        pltpu.sync_copy(out_v, out_hbm.at[pl.ds(base, e_per_w)])

    return k(src_idx, dst_idx, z)


def kernel(z, edge_index):
    idx = edge_index.astype(jnp.int32)
    return _run(idx[0], idx[1], z, idx.shape[1])


# speedup vs baseline: 4.9531x; 1.4409x over previous
import functools

import jax
import jax.numpy as jnp
from jax import lax
from jax.experimental import pallas as pl
from jax.experimental.pallas import tpu as pltpu
from jax.experimental.pallas import tpu_sc as plsc

D = 128
L = 16
NC = 2
NS = 16
NW = NC * NS
C = 80


@functools.partial(jax.jit, static_argnums=(3,))
def _run(src_idx, dst_idx, z, E):
    e_per_w = E // NW
    n_chunks = e_per_w // C
    mesh = plsc.VectorSubcoreMesh(core_axis_name="c", subcore_axis_name="s")

    @functools.partial(
        pl.kernel,
        mesh=mesh,
        compiler_params=pltpu.CompilerParams(needs_layout_passes=False),
        out_type=jax.ShapeDtypeStruct((E,), jnp.float32),
        scratch_types=[
            pltpu.VMEM((e_per_w,), jnp.int32),
            pltpu.VMEM((e_per_w,), jnp.int32),
            pltpu.VMEM((C, D), jnp.float32),
            pltpu.VMEM((C, D), jnp.float32),
            pltpu.VMEM((C, D), jnp.float32),
            pltpu.VMEM((C, D), jnp.float32),
            pltpu.VMEM((e_per_w,), jnp.float32),
            pltpu.SemaphoreType.DMA,
            pltpu.SemaphoreType.DMA,
        ],
    )
    def k(src_hbm, dst_hbm, z_hbm, out_hbm, idx_s, idx_d, rs0, rd0, rs1, rd1,
          out_v, sem0, sem1):
        cid = lax.axis_index("c")
        sid = lax.axis_index("s")
        wid = sid * NC + cid
        base = wid * e_per_w

        pltpu.sync_copy(src_hbm.at[pl.ds(base, e_per_w)], idx_s)
        pltpu.sync_copy(dst_hbm.at[pl.ds(base, e_per_w)], idx_d)

        bufs = ((rs0, rd0, sem0), (rs1, rd1, sem1))
        lane = lax.iota(jnp.int32, L)

        def issue(c, b):
            rs, rd, sem = bufs[b]
            off = c * C
            pltpu.async_copy(z_hbm.at[idx_s.at[pl.ds(off, C)]], rs, sem)
            pltpu.async_copy(z_hbm.at[idx_d.at[pl.ds(off, C)]], rd, sem)

        def wait(b):
            rs, rd, sem = bufs[b]
            pltpu.make_async_copy(z_hbm.at[idx_s.at[pl.ds(0, C)]], rs,
                                  sem).wait()
            pltpu.make_async_copy(z_hbm.at[idx_d.at[pl.ds(0, C)]], rd,
                                  sem).wait()

        def compute(c, b):
            rs, rd, _ = bufs[b]
            off = c * C

            def group_body(g, carry2):
                e0 = g * L
                res = jnp.zeros((L,), jnp.float32)
                for j in range(L):
                    e = e0 + j
                    acc = jnp.zeros((L,), jnp.float32)
                    for kk in range(D // L):
                        sv = rs[e, pl.ds(kk * L, L)]
                        dv = rd[e, pl.ds(kk * L, L)]
                        acc = acc + sv * dv
                    dot = jnp.sum(acc)
                    res = jnp.where(lane == j, dot, res)
                sig = 1.0 / (1.0 + jnp.exp(-res))
                out_v[pl.ds(off + g * L, L)] = sig
                return carry2

            lax.fori_loop(0, C // L, group_body, 0)

        issue(0, 0)

        def pair_body(t, carry):
            c0 = t * 2
            issue(c0 + 1, 1)
            wait(0)
            compute(c0, 0)
            issue(c0 + 2, 0)
            wait(1)
            compute(c0 + 1, 1)
            return carry

        lax.fori_loop(0, (n_chunks - 1) // 2, pair_body, 0)
        wait(0)
        compute(n_chunks - 1, 0)
        pltpu.sync_copy(out_v, out_hbm.at[pl.ds(base, e_per_w)])

    return k(src_idx, dst_idx, z)


def kernel(z, edge_index):
    idx = edge_index.astype(jnp.int32)
    return _run(idx[0], idx[1], z, idx.shape[1])


# speedup vs baseline: 7.5850x; 1.5314x over previous
import functools

import jax
import jax.numpy as jnp
from jax import lax
from jax.experimental import pallas as pl
from jax.experimental.pallas import tpu as pltpu
from jax.experimental.pallas import tpu_sc as plsc

D = 128
L = 16
NC = 2
NS = 16
NW = NC * NS
C = 40
B = 1000


@functools.partial(jax.jit, static_argnums=(3,))
def _run(src_idx, dst_idx, z, E):
    e_per_w = E // NW
    n_chunks = e_per_w // C
    mesh = plsc.VectorSubcoreMesh(core_axis_name="c", subcore_axis_name="s")

    @functools.partial(
        pl.kernel,
        mesh=mesh,
        compiler_params=pltpu.CompilerParams(needs_layout_passes=False),
        out_type=jax.ShapeDtypeStruct((E,), jnp.float32),
        scratch_types=[
            pltpu.VMEM_SHARED((10000, D), jnp.float32),
            pltpu.VMEM((B,), jnp.int32),
            pltpu.VMEM((B,), jnp.int32),
            pltpu.VMEM((C, D), jnp.float32),
            pltpu.VMEM((C, D), jnp.float32),
            pltpu.VMEM((C, D), jnp.float32),
            pltpu.VMEM((C, D), jnp.float32),
            pltpu.VMEM((B,), jnp.float32),
            pltpu.SemaphoreType.DMA,
            pltpu.SemaphoreType.DMA,
        ],
    )
    def k(src_hbm, dst_hbm, z_hbm, out_hbm, z_sh, idx_s, idx_d, rs0, rd0,
          rs1, rd1, out_b, sem0, sem1):
        cid = lax.axis_index("c")
        sid = lax.axis_index("s")
        wid = sid * NC + cid
        base = wid * e_per_w

        n_rows = 624
        pltpu.sync_copy(z_hbm.at[pl.ds(sid * n_rows, n_rows)],
                        z_sh.at[pl.ds(sid * n_rows, n_rows)])

        @pl.when(sid == NS - 1)
        def _stage_tail():
            pltpu.sync_copy(z_hbm.at[pl.ds(NS * n_rows, 10000 - NS * n_rows)],
                            z_sh.at[pl.ds(NS * n_rows, 10000 - NS * n_rows)])

        plsc.subcore_barrier()

        bufs = ((rs0, rd0, sem0), (rs1, rd1, sem1))
        lane = lax.iota(jnp.int32, L)
        cpb = B // C

        def issue(c, b):
            rs, rd, sem = bufs[b]
            off = c * C
            pltpu.async_copy(z_sh.at[idx_s.at[pl.ds(off, C)]], rs, sem)
            pltpu.async_copy(z_sh.at[idx_d.at[pl.ds(off, C)]], rd, sem)

        def wait(b):
            rs, rd, sem = bufs[b]
            pltpu.make_async_copy(z_hbm.at[idx_s.at[pl.ds(0, C)]], rs,
                                  sem).wait()
            pltpu.make_async_copy(z_hbm.at[idx_d.at[pl.ds(0, C)]], rd,
                                  sem).wait()

        def compute(c, b):
            rs, rd, _ = bufs[b]
            off = c * C

            def group_body(g, carry2):
                e0 = g * L
                res = jnp.zeros((L,), jnp.float32)
                for j in range(L):
                    e = e0 + j
                    acc = jnp.zeros((L,), jnp.float32)
                    for kk in range(D // L):
                        sv = rs[e, pl.ds(kk * L, L)]
                        dv = rd[e, pl.ds(kk * L, L)]
                        acc = acc + sv * dv
                    dot = jnp.sum(acc)
                    res = jnp.where(lane == j, dot, res)
                sig = 1.0 / (1.0 + jnp.exp(-res))
                out_b[pl.ds(off + g * L, L)] = sig
                return carry2

            lax.fori_loop(0, C // L, group_body, 0)

        def block_body(blk, carry):
            bbase = base + blk * B
            pltpu.sync_copy(src_hbm.at[pl.ds(bbase, B)], idx_s)
            pltpu.sync_copy(dst_hbm.at[pl.ds(bbase, B)], idx_d)
            issue(0, 0)

            def pair_body(t, carry2):
                c0 = t * 2
                issue(c0 + 1, 1)
                wait(0)
                compute(c0, 0)
                issue(c0 + 2, 0)
                wait(1)
                compute(c0 + 1, 1)
                return carry2

            lax.fori_loop(0, (cpb - 1) // 2, pair_body, 0)
            wait(0)
            compute(cpb - 1, 0)
            pltpu.sync_copy(out_b, out_hbm.at[pl.ds(bbase, B)])
            return carry

        lax.fori_loop(0, e_per_w // B, block_body, 0)

    return k(src_idx, dst_idx, z)


def kernel(z, edge_index):
    idx = edge_index.astype(jnp.int32)
    return _run(idx[0], idx[1], z, idx.shape[1])
